# Initial kernel scaffold; baseline (speedup 1.0000x reference)
import jax
import jax.numpy as jnp
from jax.experimental import pallas as pl


def kernel(x, edge_index, edge_attr, W1, b1, W2, b2):
    raise NotImplementedError("write your pallas kernel here")


# speedup vs baseline: 2.7132x; 2.7132x over previous
import functools

import jax
import jax.numpy as jnp
from jax import lax
from jax.experimental import pallas as pl
from jax.experimental.pallas import tpu as pltpu
from jax.experimental.pallas import tpu_sc as plsc

N = 10000
E = 160000
D = 256
HALF = 128
NC = 2
NS = 16
EC = E // NS
G = 80
SEC = 5
BPS = EC // (SEC * G)
NP = 10240
RPT = NP // NS


def _sc_aggregate(x2, ea2, xidx5, eaidx5, dst4):
    mesh = plsc.VectorSubcoreMesh(
        core_axis_name="c", subcore_axis_name="s",
        num_cores=NC, num_subcores=NS)

    @functools.partial(
        pl.kernel,
        out_type=jax.ShapeDtypeStruct((NC * NP, HALF), jnp.float32),
        mesh=mesh,
        scratch_types=[
            pltpu.VMEM_SHARED((NP, HALF), jnp.float32),
            pltpu.VMEM((BPS, G), jnp.int32),
            pltpu.VMEM((BPS, G), jnp.int32),
            pltpu.VMEM((BPS, G), jnp.int32),
            pltpu.VMEM((G, HALF), jnp.float32),
            pltpu.VMEM((G, HALF), jnp.float32),
            pltpu.VMEM((G, HALF), jnp.float32),
            pltpu.SemaphoreType.DMA,
            pltpu.SemaphoreType.DMA,
        ],
        compiler_params=pltpu.CompilerParams(use_tc_tiling_on_sc=False),
    )
    def k(x2_hbm, ea2_hbm, xidx_hbm, eaidx_hbm, dst_hbm, out_hbm,
          acc, xidx, eaidx, dsti, xrows, earows, msg,
          sem_x, sem_e):
        c = lax.axis_index("c")
        s = lax.axis_index("s")
        base = c * NP + s * RPT

        pltpu.sync_copy(x2_hbm.at[pl.ds(base, RPT)],
                        acc.at[pl.ds(s * RPT, RPT)])
        plsc.subcore_barrier()

        def section(sec, _):
            pltpu.sync_copy(xidx_hbm.at[c, s, sec], xidx)
            pltpu.sync_copy(eaidx_hbm.at[c, s, sec], eaidx)
            pltpu.sync_copy(dst_hbm.at[s, sec], dsti)

            def step(b, _):
                dx = pltpu.async_copy(x2_hbm.at[xidx.at[b]], xrows, sem_x)
                de = pltpu.async_copy(ea2_hbm.at[eaidx.at[b]], earows, sem_e)
                dx.wait()
                de.wait()

                def comp(e, _):
                    for kq in range(HALF // 16):
                        sl = pl.ds(kq * 16, 16)
                        msg[e, sl] = jnp.maximum(
                            xrows[e, sl] + earows[e, sl], 0.0)
                    return 0
                lax.fori_loop(0, G, comp, 0)

                pltpu.sync_copy(msg, acc.at[dsti.at[b]], add=True)
                return 0
            lax.fori_loop(0, BPS, step, 0)
            return 0
        lax.fori_loop(0, SEC, section, 0)

        plsc.subcore_barrier()
        pltpu.sync_copy(acc.at[pl.ds(s * RPT, RPT)],
                        out_hbm.at[pl.ds(base, RPT)])

    return k(x2, ea2, xidx5, eaidx5, dst4)


def _tc_mlp(h2, W1, b1, W2, b2):
    BM = 1000

    def body(h_ref, w1_ref, b1_ref, w2_ref, b2_ref, o_ref):
        h = jnp.dot(h_ref[0], w1_ref[:HALF, :],
                    preferred_element_type=jnp.float32)
        h = h + jnp.dot(h_ref[1], w1_ref[HALF:, :],
                        preferred_element_type=jnp.float32)
        h = jnp.maximum(h + b1_ref[0], 0.0)
        o_ref[...] = jnp.dot(h, w2_ref[...],
                             preferred_element_type=jnp.float32) + b2_ref[0]

    return pl.pallas_call(
        body,
        grid=(N // BM,),
        in_specs=[
            pl.BlockSpec((2, BM, HALF), lambda i: (0, i, 0)),
            pl.BlockSpec((D, D), lambda i: (0, 0)),
            pl.BlockSpec((1, D), lambda i: (0, 0)),
            pl.BlockSpec((D, D), lambda i: (0, 0)),
            pl.BlockSpec((1, D), lambda i: (0, 0)),
        ],
        out_specs=pl.BlockSpec((BM, D), lambda i: (i, 0)),
        out_shape=jax.ShapeDtypeStruct((N, D), jnp.float32),
    )(h2, W1, b1.reshape(1, D), W2, b2.reshape(1, D))


def kernel(x, edge_index, edge_attr, W1, b1, W2, b2):
    src = edge_index[0].astype(jnp.int32)
    dst = edge_index[1].astype(jnp.int32)
    pad = jnp.zeros((NP - N, HALF), jnp.float32)
    x2 = jnp.concatenate([x[:, :HALF], pad, x[:, HALF:], pad], axis=0)
    ea2 = edge_attr.reshape(2 * E, HALF)
    xidx5 = (src[None, :] + jnp.array([[0], [NP]], jnp.int32)
             ).reshape(NC, NS, SEC, BPS, G)
    e2 = jnp.arange(E, dtype=jnp.int32) * 2
    eaidx5 = (e2[None, :] + jnp.array([[0], [1]], jnp.int32)
              ).reshape(NC, NS, SEC, BPS, G)
    dst4 = dst.reshape(NS, SEC, BPS, G)
    h = _sc_aggregate(x2, ea2, xidx5, eaidx5, dst4)
    h2 = h.reshape(NC, NP, HALF)[:, :N, :]
    return _tc_mlp(h2, W1, b1, W2, b2)
